# Initial kernel scaffold; baseline (speedup 1.0000x reference)
import jax
import jax.numpy as jnp
from jax.experimental import pallas as pl


def kernel(i_feats, targets, features):
    raise NotImplementedError("write your pallas kernel here")


# speedup vs baseline: 1.1518x; 1.1518x over previous
import jax
import jax.numpy as jnp
from jax import lax
from jax.experimental import pallas as pl
from jax.experimental.pallas import tpu as pltpu
from jax.experimental.pallas import tpu_sc as plsc

BATCH = 1024
NUM_FEATURES = 768
MARGIN = 0.1
TAU = 0.02

_NC, _NS = 2, 16
_NW = _NC * _NS
_ROWS_PER_W = BATCH // _NW


def _gather_body(idx_hbm, table_hbm, out_hbm, idx_v, rows_v, sem):
    wid = lax.axis_index("s") * _NC + lax.axis_index("c")
    base = wid * _ROWS_PER_W
    pltpu.sync_copy(idx_hbm.at[pl.ds(base, _ROWS_PER_W)], idx_v)
    pltpu.async_copy(table_hbm.at[idx_v], rows_v, sem).wait()
    pltpu.sync_copy(rows_v, out_hbm.at[pl.ds(base, _ROWS_PER_W)])


def _sc_gather(targets, features):
    mesh = plsc.VectorSubcoreMesh(core_axis_name="c", subcore_axis_name="s")
    k = pl.kernel(
        _gather_body,
        mesh=mesh,
        out_type=jax.ShapeDtypeStruct((BATCH, NUM_FEATURES), jnp.float32),
        scratch_types=[
            pltpu.VMEM((_ROWS_PER_W,), jnp.int32),
            pltpu.VMEM((_ROWS_PER_W, NUM_FEATURES), jnp.float32),
            pltpu.SemaphoreType.DMA,
        ],
    )
    return k(targets.astype(jnp.int32), features)


def _loss_body(x_ref, cl_ref, trow_ref, tcol_ref, out_ref):
    x = x_ref[...]
    n = jnp.sqrt(jnp.sum(x * x, axis=1, keepdims=True))
    xi = x / jnp.maximum(n, 1e-12)
    cl = cl_ref[...]
    scores = lax.dot_general(
        xi, cl, (((1,), (1,)), ((), ())), preferred_element_type=jnp.float32
    )
    labels = tcol_ref[...] == trow_ref[...]
    neg_inf = jnp.float32(-1e30)
    s_tau = scores * jnp.float32(1.0 / TAU)
    masked = jnp.where(labels, s_tau, neg_inf)

    m1 = jnp.max(masked, axis=1, keepdims=True)
    e1 = jnp.where(labels, jnp.exp(s_tau - m1), 0.0)
    pos1 = jnp.sum(e1 * scores, axis=1, keepdims=True) / jnp.sum(
        e1, axis=1, keepdims=True
    )
    neg1 = jnp.max(jnp.where(labels, neg_inf, scores), axis=1, keepdims=True)
    c1 = jnp.maximum(MARGIN + neg1 - pos1, 0.0)

    m2 = jnp.max(masked, axis=0, keepdims=True)
    e2 = jnp.where(labels, jnp.exp(s_tau - m2), 0.0)
    pos2 = jnp.sum(e2 * scores, axis=0, keepdims=True) / jnp.sum(
        e2, axis=0, keepdims=True
    )
    neg2 = jnp.max(jnp.where(labels, neg_inf, scores), axis=0, keepdims=True)
    c2 = jnp.maximum(MARGIN + neg2 - pos2, 0.0)

    tri = jnp.sum(c1) + jnp.sum(c2)
    center = 1.0 - jnp.sum(scores) * jnp.float32(1.0 / (BATCH * BATCH))
    out_ref[0, 0] = tri + 0.08 * center


def _tc_loss(i_feats, cl, targets):
    t = targets.astype(jnp.int32)
    out = pl.pallas_call(
        _loss_body,
        out_shape=jax.ShapeDtypeStruct((1, 1), jnp.float32),
        out_specs=pl.BlockSpec(memory_space=pltpu.SMEM),
    )(i_feats, cl, t.reshape(1, BATCH), t.reshape(BATCH, 1))
    return out[0, 0]


def kernel(i_feats, targets, features):
    cl = _sc_gather(targets, features)
    return _tc_loss(i_feats, cl, targets)
